# Initial kernel scaffold; baseline (speedup 1.0000x reference)
import jax
import jax.numpy as jnp
from jax.experimental import pallas as pl


def kernel(trg_seq_embedded, ext_src_seq, init_states, encoder_outputs, encoder_mask, W_enc, b_enc, W_red, b_red, W_ih, W_hh, b_ih, b_hh, W_cat, b_cat, W_log, b_log):
    raise NotImplementedError("write your pallas kernel here")


# speedup vs baseline: 1.8798x; 1.8798x over previous
import jax
import jax.numpy as jnp
from jax.experimental import pallas as pl
from jax.experimental.pallas import tpu as pltpu

NEG_INF = 1000000000000.0
_V, _H, _E, _B, _T, _S, _OOV = 32000, 512, 300, 32, 32, 400, 50
_VO = _V + _OOV
_VP = 32768
_BV = 2048
_NJ = _VP // _BV
_RT = 16
_NR = _T // _RT
_SCH = 100


def _mem_body(enc_ref, w_ref, b_ref, out_ref):
    out_ref[...] = jax.lax.dot_general(
        enc_ref[...], w_ref[...], (((1,), (1,)), ((), ())),
        preferred_element_type=jnp.float32) + b_ref[...]


def _dec_body(emb_ref, init_ref, mem_ref, mask_ref,
              wred_ref, bred_ref, wih_ref, bih_ref, whh_ref, bhh_ref,
              wcat_ref, bcat_ref,
              attn_out, cov_out, energy_out, x_out,
              h_s, c_s, ctx_s, cov_s):
    t = pl.program_id(0)

    @pl.when(t == 0)
    def _():
        h_s[...] = init_ref[...]
        c_s[...] = init_ref[...]
        ctx_s[...] = jnp.zeros_like(ctx_s)
        cov_s[...] = jnp.zeros_like(cov_s)

    emb = emb_ref[0]
    cat1 = jnp.concatenate([emb, ctx_s[...]], axis=1)
    x = jax.lax.dot_general(cat1, wred_ref[...], (((1,), (1,)), ((), ())),
                            preferred_element_type=jnp.float32) + bred_ref[...]
    gates = (jax.lax.dot_general(x, wih_ref[...], (((1,), (1,)), ((), ())),
                                 preferred_element_type=jnp.float32)
             + bih_ref[...]
             + jax.lax.dot_general(h_s[...], whh_ref[...],
                                   (((1,), (1,)), ((), ())),
                                   preferred_element_type=jnp.float32)
             + bhh_ref[...])
    gi = gates[:, 0:_H]
    gf = gates[:, _H:2 * _H]
    gg = gates[:, 2 * _H:3 * _H]
    go = gates[:, 3 * _H:4 * _H]
    c_new = jax.nn.sigmoid(gf) * c_s[...] + jax.nn.sigmoid(gi) * jnp.tanh(gg)
    h_new = jax.nn.sigmoid(go) * jnp.tanh(c_new)

    eparts = []
    for c0 in range(0, _S, _SCH):
        chunk = mem_ref[:, c0:c0 + _SCH, :]
        eparts.append(jnp.sum(chunk * h_new[:, None, :], axis=-1))
    energy = jnp.concatenate(eparts, axis=1)
    energy = jnp.where(mask_ref[...] == 0, -NEG_INF, energy)
    m = jnp.max(energy, axis=-1, keepdims=True)
    p = jnp.exp(energy - m)
    attn = p / jnp.sum(p, axis=-1, keepdims=True)

    ctx_rows = []
    for b in range(_B):
        ctx_rows.append(jnp.dot(attn[b:b + 1, :], mem_ref[b],
                                preferred_element_type=jnp.float32))
    ctx = jnp.concatenate(ctx_rows, axis=0)

    cat2 = jnp.concatenate([h_new, ctx], axis=1)
    li = jnp.tanh(jax.lax.dot_general(cat2, wcat_ref[...],
                                      (((1,), (1,)), ((), ())),
                                      preferred_element_type=jnp.float32)
                  + bcat_ref[...])

    attn_out[0] = attn
    cov_out[0] = cov_s[...]
    energy_out[0] = energy
    x_out[0] = li
    h_s[...] = h_new
    c_s[...] = c_new
    ctx_s[...] = ctx
    cov_s[...] = cov_s[...] + attn


def _proj_body(idx_ref, idxt_ref, x_ref, w_ref, b_ref, e_ref, out_ref, wf_s):
    j = pl.program_id(1)

    @pl.when(j == 0)
    def _():
        e3 = e_ref[...]
        rows = jax.lax.broadcasted_iota(jnp.int32, (_S, _S), 0)
        cols = jax.lax.broadcasted_iota(jnp.int32, (_S, _S), 1)
        earlier = rows < cols
        for b in range(_B):
            idx_col = idxt_ref[:, b:b + 1]
            idx_row = idx_ref[b:b + 1, :]
            same = idx_col == idx_row
            eb = e3[:, b, :]
            wm = jnp.max(jnp.where(same[None, :, :], eb[:, None, :],
                                   -NEG_INF), axis=-1)
            ndup = jnp.sum(jnp.where(same & earlier, 1, 0),
                           axis=0, keepdims=True)
            first = ndup == 0
            wf_s[:, b, :] = jnp.where(first & (wm != -NEG_INF), wm, 0.0)

    logit = jax.lax.dot_general(
        x_ref[...].reshape(_RT * _B, _H), w_ref[...], (((1,), (1,)), ((), ())),
        preferred_element_type=jnp.float32) + b_ref[...]
    logit3 = logit.reshape(_RT, _B, _BV)

    col_ids = j * _BV + jax.lax.broadcasted_iota(jnp.int32, (1, _BV), 1)
    for b in range(_B):
        onehot = (idxt_ref[:, b:b + 1] == col_ids).astype(jnp.float32)
        contrib = jnp.dot(wf_s[:, b, :], onehot,
                          preferred_element_type=jnp.float32)
        val = logit3[:, b, :] + contrib
        out_ref[:, b, :] = jnp.where(val == 0.0, -NEG_INF, val)


def kernel(trg_seq_embedded, ext_src_seq, init_states, encoder_outputs,
           encoder_mask, W_enc, b_enc, W_red, b_red, W_ih, W_hh, b_ih, b_hh,
           W_cat, b_cat, W_log, b_log):
    f32 = jnp.float32
    whole = lambda: pl.BlockSpec(memory_space=pltpu.VMEM)

    enc2 = encoder_outputs.reshape(_B * _S, _H)
    n_mb = 8
    rb = (_B * _S) // n_mb
    memories = pl.pallas_call(
        _mem_body,
        grid=(n_mb,),
        in_specs=[
            pl.BlockSpec((rb, _H), lambda i: (i, 0)),
            whole(),
            whole(),
        ],
        out_specs=pl.BlockSpec((rb, _H), lambda i: (i, 0)),
        out_shape=jax.ShapeDtypeStruct((_B * _S, _H), f32),
        compiler_params=pltpu.CompilerParams(
            dimension_semantics=("parallel",),
            vmem_limit_bytes=40 * 1024 * 1024,
        ),
    )(enc2, W_enc, b_enc.reshape(1, _H))
    mem3 = memories.reshape(_B, _S, _H)

    embT = jnp.swapaxes(trg_seq_embedded, 0, 1)
    out_shapes = [
        jax.ShapeDtypeStruct((_T, _B, _S), f32),
        jax.ShapeDtypeStruct((_T, _B, _S), f32),
        jax.ShapeDtypeStruct((_T, _B, _S), f32),
        jax.ShapeDtypeStruct((_T, _B, _H), f32),
    ]
    step_spec = lambda shp: pl.BlockSpec(shp, lambda t: (t, 0, 0))
    attns, covs, energies, x3 = pl.pallas_call(
        _dec_body,
        grid=(_T,),
        in_specs=[
            step_spec((1, _B, _E)),
            whole(),
            whole(),
            whole(),
            whole(), whole(), whole(), whole(),
            whole(), whole(), whole(), whole(),
        ],
        out_specs=[
            step_spec((1, _B, _S)),
            step_spec((1, _B, _S)),
            step_spec((1, _B, _S)),
            step_spec((1, _B, _H)),
        ],
        out_shape=out_shapes,
        scratch_shapes=[
            pltpu.VMEM((_B, _H), f32),
            pltpu.VMEM((_B, _H), f32),
            pltpu.VMEM((_B, _H), f32),
            pltpu.VMEM((_B, _S), f32),
        ],
        compiler_params=pltpu.CompilerParams(
            dimension_semantics=("arbitrary",),
            vmem_limit_bytes=60 * 1024 * 1024,
        ),
    )(embT, init_states[0], mem3, encoder_mask,
      W_red, b_red.reshape(1, _E), W_ih, b_ih.reshape(1, 4 * _H),
      W_hh, b_hh.reshape(1, 4 * _H), W_cat, b_cat.reshape(1, _H))

    w_pad = jnp.pad(W_log, ((0, _VP - _V), (0, 0)))
    b_pad = jnp.pad(b_log, (0, _VP - _V)).reshape(1, _VP)
    logits = pl.pallas_call(
        _proj_body,
        grid=(_NR, _NJ),
        in_specs=[
            whole(),
            whole(),
            pl.BlockSpec((_RT, _B, _H), lambda r, j: (r, 0, 0)),
            pl.BlockSpec((_BV, _H), lambda r, j: (j, 0)),
            pl.BlockSpec((1, _BV), lambda r, j: (0, j)),
            pl.BlockSpec((_RT, _B, _S), lambda r, j: (r, 0, 0)),
        ],
        out_specs=pl.BlockSpec((_RT, _B, _BV), lambda r, j: (r, 0, j)),
        out_shape=jax.ShapeDtypeStruct((_T, _B, _VO), f32),
        scratch_shapes=[pltpu.VMEM((_RT, _B, _S), f32)],
        compiler_params=pltpu.CompilerParams(
            dimension_semantics=("parallel", "arbitrary"),
            vmem_limit_bytes=60 * 1024 * 1024,
        ),
    )(ext_src_seq, ext_src_seq.T, x3, w_pad, b_pad, energies)

    return logits, attns, covs, energies


# speedup vs baseline: 3.1997x; 1.7022x over previous
import jax
import jax.numpy as jnp
from jax.experimental import pallas as pl
from jax.experimental.pallas import tpu as pltpu

NEG_INF = 1000000000000.0
_V, _H, _E, _B, _T, _S, _OOV = 32000, 512, 300, 32, 32, 400, 50
_VO = _V + _OOV
_VP = 32768
_BV = 2048
_NJ = _VP // _BV
_BH = 16
_NC = _B // _BH
_SCH = 100


def _mem_body(enc_ref, w_ref, b_ref, out_ref):
    out_ref[...] = jax.lax.dot_general(
        enc_ref[...], w_ref[...], (((1,), (1,)), ((), ())),
        preferred_element_type=jnp.float32) + b_ref[...]


def _dec_body(emb_ref, init_ref, mem_ref, mask_ref,
              wred_ref, bred_ref, wih_ref, bih_ref, whh_ref, bhh_ref,
              wcat_ref, bcat_ref,
              attn_out, cov_out, energy_out, x_out,
              h_s, c_s, ctx_s, cov_s):
    t = pl.program_id(1)

    @pl.when(t == 0)
    def _():
        h_s[...] = init_ref[...]
        c_s[...] = init_ref[...]
        ctx_s[...] = jnp.zeros_like(ctx_s)
        cov_s[...] = jnp.zeros_like(cov_s)

    emb = emb_ref[0]
    cat1 = jnp.concatenate([emb, ctx_s[...]], axis=1)
    x = jax.lax.dot_general(cat1, wred_ref[...], (((1,), (1,)), ((), ())),
                            preferred_element_type=jnp.float32) + bred_ref[...]
    gates = (jax.lax.dot_general(x, wih_ref[...], (((1,), (1,)), ((), ())),
                                 preferred_element_type=jnp.float32)
             + bih_ref[...]
             + jax.lax.dot_general(h_s[...], whh_ref[...],
                                   (((1,), (1,)), ((), ())),
                                   preferred_element_type=jnp.float32)
             + bhh_ref[...])
    gi = gates[:, 0:_H]
    gf = gates[:, _H:2 * _H]
    gg = gates[:, 2 * _H:3 * _H]
    go = gates[:, 3 * _H:4 * _H]
    c_new = jax.nn.sigmoid(gf) * c_s[...] + jax.nn.sigmoid(gi) * jnp.tanh(gg)
    h_new = jax.nn.sigmoid(go) * jnp.tanh(c_new)

    e_rows = []
    for b in range(_BH):
        e_rows.append(jax.lax.dot_general(
            h_new[b:b + 1, :], mem_ref[b], (((1,), (1,)), ((), ())),
            preferred_element_type=jnp.float32))
    energy = jnp.concatenate(e_rows, axis=0)
    energy = jnp.where(mask_ref[...] == 0, -NEG_INF, energy)
    m = jnp.max(energy, axis=-1, keepdims=True)
    p = jnp.exp(energy - m)
    attn = p / jnp.sum(p, axis=-1, keepdims=True)

    ctx_rows = []
    for b in range(_BH):
        ctx_rows.append(jnp.dot(attn[b:b + 1, :], mem_ref[b],
                                preferred_element_type=jnp.float32))
    ctx = jnp.concatenate(ctx_rows, axis=0)

    cat2 = jnp.concatenate([h_new, ctx], axis=1)
    li = jnp.tanh(jax.lax.dot_general(cat2, wcat_ref[...],
                                      (((1,), (1,)), ((), ())),
                                      preferred_element_type=jnp.float32)
                  + bcat_ref[...])

    attn_out[0] = attn
    cov_out[0] = cov_s[...]
    energy_out[0] = energy
    x_out[0] = li
    h_s[...] = h_new
    c_s[...] = c_new
    ctx_s[...] = ctx
    cov_s[...] = cov_s[...] + attn


def _wf_body(idx_ref, idxt_ref, e_ref, wf_ref):
    rows = jax.lax.broadcasted_iota(jnp.int32, (_S, _S), 0)
    cols = jax.lax.broadcasted_iota(jnp.int32, (_S, _S), 1)
    earlier = rows < cols
    for b in range(_BH):
        idx_col = idxt_ref[0][:, b:b + 1]
        idx_row = idx_ref[b:b + 1, :]
        same = idx_col == idx_row
        ndup = jnp.sum(jnp.where(same & earlier, 1, 0),
                       axis=0, keepdims=True)
        first = ndup == 0
        for t0 in range(0, _T, 8):
            eb = e_ref[t0:t0 + 8, b, :]
            wm = jnp.max(jnp.where(same[None, :, :], eb[:, None, :],
                                   -NEG_INF), axis=-1)
            wf_ref[t0:t0 + 8, b, :] = jnp.where(
                first & (wm != -NEG_INF), wm, 0.0)


def _proj_body(idxt_ref, x_ref, w_ref, b_ref, wf_ref, out_ref):
    j = pl.program_id(0)
    logit = jax.lax.dot_general(
        x_ref[...].reshape(_T * _B, _H), w_ref[...], (((1,), (1,)), ((), ())),
        preferred_element_type=jnp.float32) + b_ref[...]
    logit3 = logit.reshape(_T, _B, _BV)

    col_ids = j * _BV + jax.lax.broadcasted_iota(jnp.int32, (1, _BV), 1)
    for b in range(_B):
        onehot = (idxt_ref[:, b:b + 1] == col_ids).astype(jnp.bfloat16)
        contrib = jnp.dot(wf_ref[:, b, :].astype(jnp.bfloat16), onehot,
                          preferred_element_type=jnp.float32)
        val = logit3[:, b, :] + contrib
        out_ref[:, b, :] = jnp.where(val == 0.0, -NEG_INF, val)


def kernel(trg_seq_embedded, ext_src_seq, init_states, encoder_outputs,
           encoder_mask, W_enc, b_enc, W_red, b_red, W_ih, W_hh, b_ih, b_hh,
           W_cat, b_cat, W_log, b_log):
    f32 = jnp.float32
    whole = lambda: pl.BlockSpec(memory_space=pltpu.VMEM)

    enc2 = encoder_outputs.reshape(_B * _S, _H)
    n_mb = 8
    rb = (_B * _S) // n_mb
    memories = pl.pallas_call(
        _mem_body,
        grid=(n_mb,),
        in_specs=[
            pl.BlockSpec((rb, _H), lambda i: (i, 0)),
            whole(),
            whole(),
        ],
        out_specs=pl.BlockSpec((rb, _H), lambda i: (i, 0)),
        out_shape=jax.ShapeDtypeStruct((_B * _S, _H), f32),
        compiler_params=pltpu.CompilerParams(
            dimension_semantics=("parallel",),
            vmem_limit_bytes=40 * 1024 * 1024,
        ),
    )(enc2, W_enc, b_enc.reshape(1, _H))
    mem3 = memories.reshape(_B, _S, _H)

    embT = jnp.swapaxes(trg_seq_embedded, 0, 1)
    out_shapes = [
        jax.ShapeDtypeStruct((_T, _B, _S), f32),
        jax.ShapeDtypeStruct((_T, _B, _S), f32),
        jax.ShapeDtypeStruct((_T, _B, _S), f32),
        jax.ShapeDtypeStruct((_T, _B, _H), f32),
    ]
    step_spec = lambda d: pl.BlockSpec((1, _BH, d), lambda c, t: (t, c, 0))
    attns, covs, energies, x3 = pl.pallas_call(
        _dec_body,
        grid=(_NC, _T),
        in_specs=[
            step_spec(_E),
            pl.BlockSpec((_BH, _H), lambda c, t: (c, 0)),
            pl.BlockSpec((_BH, _S, _H), lambda c, t: (c, 0, 0)),
            pl.BlockSpec((_BH, _S), lambda c, t: (c, 0)),
            whole(), whole(), whole(), whole(),
            whole(), whole(), whole(), whole(),
        ],
        out_specs=[
            step_spec(_S),
            step_spec(_S),
            step_spec(_S),
            step_spec(_H),
        ],
        out_shape=out_shapes,
        scratch_shapes=[
            pltpu.VMEM((_BH, _H), f32),
            pltpu.VMEM((_BH, _H), f32),
            pltpu.VMEM((_BH, _H), f32),
            pltpu.VMEM((_BH, _S), f32),
        ],
        compiler_params=pltpu.CompilerParams(
            dimension_semantics=("parallel", "arbitrary"),
            vmem_limit_bytes=50 * 1024 * 1024,
        ),
    )(embT, init_states[0], mem3, encoder_mask,
      W_red, b_red.reshape(1, _E), W_ih, b_ih.reshape(1, 4 * _H),
      W_hh, b_hh.reshape(1, 4 * _H), W_cat, b_cat.reshape(1, _H))

    idx_t = ext_src_seq.T
    idx_t3 = jnp.swapaxes(ext_src_seq.reshape(_NC, _BH, _S), 1, 2)
    wf = pl.pallas_call(
        _wf_body,
        grid=(_NC,),
        in_specs=[
            pl.BlockSpec((_BH, _S), lambda c: (c, 0)),
            pl.BlockSpec((1, _S, _BH), lambda c: (c, 0, 0)),
            pl.BlockSpec((_T, _BH, _S), lambda c: (0, c, 0)),
        ],
        out_specs=pl.BlockSpec((_T, _BH, _S), lambda c: (0, c, 0)),
        out_shape=jax.ShapeDtypeStruct((_T, _B, _S), f32),
        compiler_params=pltpu.CompilerParams(
            dimension_semantics=("parallel",),
            vmem_limit_bytes=50 * 1024 * 1024,
        ),
    )(ext_src_seq, idx_t3, energies)

    w_pad = jnp.pad(W_log, ((0, _VP - _V), (0, 0))).astype(jnp.bfloat16)
    b_pad = jnp.pad(b_log, (0, _VP - _V)).reshape(1, _VP)
    x3b = x3.astype(jnp.bfloat16)
    logits = pl.pallas_call(
        _proj_body,
        grid=(_NJ,),
        in_specs=[
            whole(),
            whole(),
            pl.BlockSpec((_BV, _H), lambda j: (j, 0)),
            pl.BlockSpec((1, _BV), lambda j: (0, j)),
            whole(),
        ],
        out_specs=pl.BlockSpec((_T, _B, _BV), lambda j: (0, 0, j)),
        out_shape=jax.ShapeDtypeStruct((_T, _B, _VO), f32),
        compiler_params=pltpu.CompilerParams(
            dimension_semantics=("parallel",),
            vmem_limit_bytes=60 * 1024 * 1024,
        ),
    )(idx_t, x3b, w_pad, b_pad, wf)

    return logits, attns, covs, energies


# speedup vs baseline: 3.2077x; 1.0025x over previous
import jax
import jax.numpy as jnp
from jax.experimental import pallas as pl
from jax.experimental.pallas import tpu as pltpu

NEG_INF = 1000000000000.0
_V, _H, _E, _B, _T, _S, _OOV = 32000, 512, 300, 32, 32, 400, 50
_VO = _V + _OOV
_BV = 1280
_NJ = _V // _BV
_BH = 16
_NC = _B // _BH
_SCH = 100


def _mem_body(enc_ref, w_ref, b_ref, out_ref):
    out_ref[...] = (jax.lax.dot_general(
        enc_ref[...], w_ref[...], (((1,), (1,)), ((), ())),
        preferred_element_type=jnp.float32) + b_ref[...]).astype(jnp.bfloat16)


def _dec_body(emb_ref, init_ref, mem_ref, mask_ref,
              wred_ref, bred_ref, wih_ref, bih_ref, whh_ref, bhh_ref,
              wcat_ref, bcat_ref,
              attn_out, cov_out, energy_out, x_out,
              h_s, c_s, ctx_s, cov_s):
    t = pl.program_id(1)

    @pl.when(t == 0)
    def _():
        h_s[...] = init_ref[...]
        c_s[...] = init_ref[...]
        ctx_s[...] = jnp.zeros_like(ctx_s)
        cov_s[...] = jnp.zeros_like(cov_s)

    emb = emb_ref[0]
    cat1 = jnp.concatenate([emb, ctx_s[...]], axis=1)
    x = jax.lax.dot_general(cat1, wred_ref[...], (((1,), (1,)), ((), ())),
                            preferred_element_type=jnp.float32) + bred_ref[...]
    gates = (jax.lax.dot_general(x, wih_ref[...], (((1,), (1,)), ((), ())),
                                 preferred_element_type=jnp.float32)
             + bih_ref[...]
             + jax.lax.dot_general(h_s[...], whh_ref[...],
                                   (((1,), (1,)), ((), ())),
                                   preferred_element_type=jnp.float32)
             + bhh_ref[...])
    gi = gates[:, 0:_H]
    gf = gates[:, _H:2 * _H]
    gg = gates[:, 2 * _H:3 * _H]
    go = gates[:, 3 * _H:4 * _H]
    c_new = jax.nn.sigmoid(gf) * c_s[...] + jax.nn.sigmoid(gi) * jnp.tanh(gg)
    h_new = jax.nn.sigmoid(go) * jnp.tanh(c_new)

    h_b16 = h_new.astype(jnp.bfloat16)
    e_rows = []
    for b in range(_BH):
        e_rows.append(jax.lax.dot_general(
            h_b16[b:b + 1, :], mem_ref[b], (((1,), (1,)), ((), ())),
            preferred_element_type=jnp.float32))
    energy = jnp.concatenate(e_rows, axis=0)
    energy = jnp.where(mask_ref[...] == 0, -NEG_INF, energy)
    m = jnp.max(energy, axis=-1, keepdims=True)
    p = jnp.exp(energy - m)
    attn = p / jnp.sum(p, axis=-1, keepdims=True)

    a_b16 = attn.astype(jnp.bfloat16)
    ctx_rows = []
    for b in range(_BH):
        ctx_rows.append(jnp.dot(a_b16[b:b + 1, :], mem_ref[b],
                                preferred_element_type=jnp.float32))
    ctx = jnp.concatenate(ctx_rows, axis=0)

    cat2 = jnp.concatenate([h_new, ctx], axis=1)
    li = jnp.tanh(jax.lax.dot_general(cat2, wcat_ref[...],
                                      (((1,), (1,)), ((), ())),
                                      preferred_element_type=jnp.float32)
                  + bcat_ref[...])

    attn_out[0] = attn
    cov_out[0] = cov_s[...]
    energy_out[0] = energy
    x_out[0] = li
    h_s[...] = h_new
    c_s[...] = c_new
    ctx_s[...] = ctx
    cov_s[...] = cov_s[...] + attn


def _wf_body(idx_ref, idxt_ref, e_ref, wf_ref):
    rows = jax.lax.broadcasted_iota(jnp.int32, (_S, _S), 0)
    cols = jax.lax.broadcasted_iota(jnp.int32, (_S, _S), 1)
    earlier = rows < cols
    for b in range(_BH):
        idx_col = idxt_ref[0][:, b:b + 1]
        idx_row = idx_ref[b:b + 1, :]
        same = idx_col == idx_row
        ndup = jnp.sum(jnp.where(same & earlier, 1, 0),
                       axis=0, keepdims=True)
        first = ndup == 0
        for t0 in range(0, _T, 8):
            eb = e_ref[t0:t0 + 8, b, :]
            wm = jnp.max(jnp.where(same[None, :, :], eb[:, None, :],
                                   -NEG_INF), axis=-1)
            wf_ref[t0:t0 + 8, b, :] = jnp.where(
                first & (wm != -NEG_INF), wm, 0.0)


def _proj_body(idxt_ref, x_ref, w_ref, b_ref, wf_ref, out_ref):
    j = pl.program_id(0)
    logit = jax.lax.dot_general(
        x_ref[...].reshape(_T * _B, _H), w_ref[...], (((1,), (1,)), ((), ())),
        preferred_element_type=jnp.float32) + b_ref[...]
    logit = jnp.where(j < _NJ, logit, 0.0)
    logit3 = logit.reshape(_T, _B, _BV)

    col_ids = j * _BV + jax.lax.broadcasted_iota(jnp.int32, (1, _BV), 1)
    for b in range(_B):
        onehot = (idxt_ref[:, b:b + 1] == col_ids).astype(jnp.bfloat16)
        contrib = jnp.dot(wf_ref[:, b, :].astype(jnp.bfloat16), onehot,
                          preferred_element_type=jnp.float32)
        val = logit3[:, b, :] + contrib
        out_ref[:, b, :] = jnp.where(val == 0.0, -NEG_INF, val)


def kernel(trg_seq_embedded, ext_src_seq, init_states, encoder_outputs,
           encoder_mask, W_enc, b_enc, W_red, b_red, W_ih, W_hh, b_ih, b_hh,
           W_cat, b_cat, W_log, b_log):
    f32 = jnp.float32
    whole = lambda: pl.BlockSpec(memory_space=pltpu.VMEM)

    enc2 = encoder_outputs.reshape(_B * _S, _H)
    n_mb = 8
    rb = (_B * _S) // n_mb
    memories = pl.pallas_call(
        _mem_body,
        grid=(n_mb,),
        in_specs=[
            pl.BlockSpec((rb, _H), lambda i: (i, 0)),
            whole(),
            whole(),
        ],
        out_specs=pl.BlockSpec((rb, _H), lambda i: (i, 0)),
        out_shape=jax.ShapeDtypeStruct((_B * _S, _H), jnp.bfloat16),
        compiler_params=pltpu.CompilerParams(
            dimension_semantics=("parallel",),
            vmem_limit_bytes=40 * 1024 * 1024,
        ),
    )(enc2, W_enc, b_enc.reshape(1, _H))
    mem3 = memories.reshape(_B, _S, _H)

    embT = jnp.swapaxes(trg_seq_embedded, 0, 1)
    out_shapes = [
        jax.ShapeDtypeStruct((_T, _B, _S), f32),
        jax.ShapeDtypeStruct((_T, _B, _S), f32),
        jax.ShapeDtypeStruct((_T, _B, _S), f32),
        jax.ShapeDtypeStruct((_T, _B, _H), f32),
    ]
    step_spec = lambda d: pl.BlockSpec((1, _BH, d), lambda c, t: (t, c, 0))
    attns, covs, energies, x3 = pl.pallas_call(
        _dec_body,
        grid=(_NC, _T),
        in_specs=[
            step_spec(_E),
            pl.BlockSpec((_BH, _H), lambda c, t: (c, 0)),
            pl.BlockSpec((_BH, _S, _H), lambda c, t: (c, 0, 0)),
            pl.BlockSpec((_BH, _S), lambda c, t: (c, 0)),
            whole(), whole(), whole(), whole(),
            whole(), whole(), whole(), whole(),
        ],
        out_specs=[
            step_spec(_S),
            step_spec(_S),
            step_spec(_S),
            step_spec(_H),
        ],
        out_shape=out_shapes,
        scratch_shapes=[
            pltpu.VMEM((_BH, _H), f32),
            pltpu.VMEM((_BH, _H), f32),
            pltpu.VMEM((_BH, _H), f32),
            pltpu.VMEM((_BH, _S), f32),
        ],
        compiler_params=pltpu.CompilerParams(
            dimension_semantics=("parallel", "arbitrary"),
            vmem_limit_bytes=50 * 1024 * 1024,
        ),
    )(embT, init_states[0], mem3, encoder_mask,
      W_red, b_red.reshape(1, _E), W_ih, b_ih.reshape(1, 4 * _H),
      W_hh, b_hh.reshape(1, 4 * _H), W_cat, b_cat.reshape(1, _H))

    idx_t = ext_src_seq.T
    idx_t3 = jnp.swapaxes(ext_src_seq.reshape(_NC, _BH, _S), 1, 2)
    wf = pl.pallas_call(
        _wf_body,
        grid=(_NC,),
        in_specs=[
            pl.BlockSpec((_BH, _S), lambda c: (c, 0)),
            pl.BlockSpec((1, _S, _BH), lambda c: (c, 0, 0)),
            pl.BlockSpec((_T, _BH, _S), lambda c: (0, c, 0)),
        ],
        out_specs=pl.BlockSpec((_T, _BH, _S), lambda c: (0, c, 0)),
        out_shape=jax.ShapeDtypeStruct((_T, _B, _S), f32),
        compiler_params=pltpu.CompilerParams(
            dimension_semantics=("parallel",),
            vmem_limit_bytes=50 * 1024 * 1024,
        ),
    )(ext_src_seq, idx_t3, energies)

    w_b16 = W_log.astype(jnp.bfloat16)
    b2 = b_log.reshape(1, _V)
    x3b = x3.astype(jnp.bfloat16)
    wclamp = lambda j: (jnp.minimum(j, _NJ - 1), 0)
    bclamp = lambda j: (0, jnp.minimum(j, _NJ - 1))
    logits = pl.pallas_call(
        _proj_body,
        grid=(_NJ + 1,),
        in_specs=[
            whole(),
            whole(),
            pl.BlockSpec((_BV, _H), wclamp),
            pl.BlockSpec((1, _BV), bclamp),
            whole(),
        ],
        out_specs=pl.BlockSpec((_T, _B, _BV), lambda j: (0, 0, j)),
        out_shape=jax.ShapeDtypeStruct((_T, _B, _VO), f32),
        compiler_params=pltpu.CompilerParams(
            dimension_semantics=("parallel",),
            vmem_limit_bytes=60 * 1024 * 1024,
        ),
    )(idx_t, x3b, w_b16, b2, wf)

    return logits, attns, covs, energies


# speedup vs baseline: 3.2166x; 1.0028x over previous
import jax
import jax.numpy as jnp
from jax.experimental import pallas as pl
from jax.experimental.pallas import tpu as pltpu

NEG_INF = 1000000000000.0
_V, _H, _E, _B, _T, _S, _OOV = 32000, 512, 300, 32, 32, 400, 50
_VO = _V + _OOV
_BV = 1280
_NJ = _V // _BV
_BH = 16
_NC = _B // _BH


def _mem_body(enc_ref, w_ref, b_ref, out_ref):
    out_ref[...] = (jax.lax.dot_general(
        enc_ref[...], w_ref[...], (((1,), (1,)), ((), ())),
        preferred_element_type=jnp.float32) + b_ref[...]).astype(jnp.bfloat16)


def _dec_body(emb_ref, init_ref, mem_ref, mask_ref,
              wred_ref, bred_ref, wih_ref, bih_ref, whh_ref, bhh_ref,
              wcat_ref, bcat_ref,
              attn_out, cov_out, energy_out, x_out,
              h_s, c_s, ctx_s, cov_s):
    t = pl.program_id(1)

    @pl.when(t == 0)
    def _():
        h_s[...] = init_ref[...]
        c_s[...] = init_ref[...]
        ctx_s[...] = jnp.zeros_like(ctx_s)
        cov_s[...] = jnp.zeros_like(cov_s)

    emb = emb_ref[0]
    cat1 = jnp.concatenate([emb, ctx_s[...]], axis=1)
    x = jax.lax.dot_general(cat1, wred_ref[...], (((1,), (1,)), ((), ())),
                            preferred_element_type=jnp.float32) + bred_ref[...]
    gates = (jax.lax.dot_general(x, wih_ref[...], (((1,), (1,)), ((), ())),
                                 preferred_element_type=jnp.float32)
             + bih_ref[...]
             + jax.lax.dot_general(h_s[...], whh_ref[...],
                                   (((1,), (1,)), ((), ())),
                                   preferred_element_type=jnp.float32)
             + bhh_ref[...])
    gi = gates[:, 0:_H]
    gf = gates[:, _H:2 * _H]
    gg = gates[:, 2 * _H:3 * _H]
    go = gates[:, 3 * _H:4 * _H]
    c_new = jax.nn.sigmoid(gf) * c_s[...] + jax.nn.sigmoid(gi) * jnp.tanh(gg)
    h_new = jax.nn.sigmoid(go) * jnp.tanh(c_new)

    h_b16 = h_new.astype(jnp.bfloat16)
    e_rows = []
    for b in range(_BH):
        e_rows.append(jax.lax.dot_general(
            h_b16[b:b + 1, :], mem_ref[b], (((1,), (1,)), ((), ())),
            preferred_element_type=jnp.float32))
    energy = jnp.concatenate(e_rows, axis=0)
    energy = jnp.where(mask_ref[...] == 0, -NEG_INF, energy)
    m = jnp.max(energy, axis=-1, keepdims=True)
    p = jnp.exp(energy - m)
    attn = p / jnp.sum(p, axis=-1, keepdims=True)

    a_b16 = attn.astype(jnp.bfloat16)
    ctx_rows = []
    for b in range(_BH):
        ctx_rows.append(jnp.dot(a_b16[b:b + 1, :], mem_ref[b],
                                preferred_element_type=jnp.float32))
    ctx = jnp.concatenate(ctx_rows, axis=0)

    cat2 = jnp.concatenate([h_new, ctx], axis=1)
    li = jnp.tanh(jax.lax.dot_general(cat2, wcat_ref[...],
                                      (((1,), (1,)), ((), ())),
                                      preferred_element_type=jnp.float32)
                  + bcat_ref[...])

    attn_out[0] = attn
    cov_out[0] = cov_s[...]
    energy_out[0] = energy
    x_out[0] = li
    h_s[...] = h_new
    c_s[...] = c_new
    ctx_s[...] = ctx
    cov_s[...] = cov_s[...] + attn


def _wf_body(idx_ref, idxt_ref, e_ref, wf_ref):
    rows = jax.lax.broadcasted_iota(jnp.int32, (_S, _S), 0)
    cols = jax.lax.broadcasted_iota(jnp.int32, (_S, _S), 1)
    earlier = rows < cols
    for b in range(_BH):
        idx_col = idxt_ref[0][:, b:b + 1]
        idx_row = idx_ref[b:b + 1, :]
        same = idx_col == idx_row
        ndup = jnp.sum(jnp.where(same & earlier, 1, 0),
                       axis=0, keepdims=True)
        first = ndup == 0
        for t0 in range(0, _T, 8):
            eb = e_ref[t0:t0 + 8, b, :]
            wm = jnp.max(jnp.where(same[None, :, :], eb[:, None, :],
                                   -NEG_INF), axis=-1)
            wf_ref[t0:t0 + 8, b, :] = jnp.where(
                first & (wm != -NEG_INF), wm, 0.0)


def _proj_body(idxt_ref, x_ref, w_ref, b_ref, wf_ref, out_ref):
    j = pl.program_id(0)
    logit = jax.lax.dot_general(
        x_ref[...].reshape(_T * _B, _H), w_ref[...], (((1,), (1,)), ((), ())),
        preferred_element_type=jnp.float32) + b_ref[...]
    logit = jnp.where(j < _NJ, logit, 0.0)
    logit3 = logit.reshape(_T, _B, _BV)

    col_ids = j * _BV + jax.lax.broadcasted_iota(jnp.int32, (1, _BV), 1)
    for b in range(_B):
        onehot = (idxt_ref[:, b:b + 1] == col_ids).astype(jnp.bfloat16)
        contrib = jnp.dot(wf_ref[:, b, :].astype(jnp.bfloat16), onehot,
                          preferred_element_type=jnp.float32)
        val = logit3[:, b, :] + contrib
        out_ref[:, b, :] = jnp.where(val == 0.0, -NEG_INF, val)


def kernel(trg_seq_embedded, ext_src_seq, init_states, encoder_outputs,
           encoder_mask, W_enc, b_enc, W_red, b_red, W_ih, W_hh, b_ih, b_hh,
           W_cat, b_cat, W_log, b_log):
    f32 = jnp.float32
    whole = lambda: pl.BlockSpec(memory_space=pltpu.VMEM)

    enc2 = encoder_outputs.reshape(_B * _S, _H)
    n_mb = 8
    rb = (_B * _S) // n_mb
    memories = pl.pallas_call(
        _mem_body,
        grid=(n_mb,),
        in_specs=[
            pl.BlockSpec((rb, _H), lambda i: (i, 0)),
            whole(),
            whole(),
        ],
        out_specs=pl.BlockSpec((rb, _H), lambda i: (i, 0)),
        out_shape=jax.ShapeDtypeStruct((_B * _S, _H), jnp.bfloat16),
        compiler_params=pltpu.CompilerParams(
            dimension_semantics=("parallel",),
            vmem_limit_bytes=40 * 1024 * 1024,
        ),
    )(enc2, W_enc, b_enc.reshape(1, _H))
    mem3 = memories.reshape(_B, _S, _H)

    embT = jnp.swapaxes(trg_seq_embedded, 0, 1)
    out_shapes = [
        jax.ShapeDtypeStruct((_T, _B, _S), f32),
        jax.ShapeDtypeStruct((_T, _B, _S), f32),
        jax.ShapeDtypeStruct((_T, _B, _S), f32),
        jax.ShapeDtypeStruct((_T, _B, _H), f32),
    ]
    step_spec = lambda d: pl.BlockSpec((1, _BH, d), lambda c, t: (t, c, 0))
    attns, covs, energies, x3 = pl.pallas_call(
        _dec_body,
        grid=(_NC, _T),
        in_specs=[
            step_spec(_E),
            pl.BlockSpec((_BH, _H), lambda c, t: (c, 0)),
            pl.BlockSpec((_BH, _S, _H), lambda c, t: (c, 0, 0)),
            pl.BlockSpec((_BH, _S), lambda c, t: (c, 0)),
            whole(), whole(), whole(), whole(),
            whole(), whole(), whole(), whole(),
        ],
        out_specs=[
            step_spec(_S),
            step_spec(_S),
            step_spec(_S),
            step_spec(_H),
        ],
        out_shape=out_shapes,
        scratch_shapes=[
            pltpu.VMEM((_BH, _H), f32),
            pltpu.VMEM((_BH, _H), f32),
            pltpu.VMEM((_BH, _H), f32),
            pltpu.VMEM((_BH, _S), f32),
        ],
        compiler_params=pltpu.CompilerParams(
            dimension_semantics=("parallel", "arbitrary"),
            vmem_limit_bytes=50 * 1024 * 1024,
        ),
    )(embT, init_states[0], mem3, encoder_mask,
      W_red, b_red.reshape(1, _E), W_ih, b_ih.reshape(1, 4 * _H),
      W_hh, b_hh.reshape(1, 4 * _H), W_cat, b_cat.reshape(1, _H))

    idx_t = ext_src_seq.T
    idx_t3 = jnp.swapaxes(ext_src_seq.reshape(_NC, _BH, _S), 1, 2)
    wf = pl.pallas_call(
        _wf_body,
        grid=(_NC,),
        in_specs=[
            pl.BlockSpec((_BH, _S), lambda c: (c, 0)),
            pl.BlockSpec((1, _S, _BH), lambda c: (c, 0, 0)),
            pl.BlockSpec((_T, _BH, _S), lambda c: (0, c, 0)),
        ],
        out_specs=pl.BlockSpec((_T, _BH, _S), lambda c: (0, c, 0)),
        out_shape=jax.ShapeDtypeStruct((_T, _B, _S), f32),
        compiler_params=pltpu.CompilerParams(
            dimension_semantics=("parallel",),
            vmem_limit_bytes=50 * 1024 * 1024,
        ),
    )(ext_src_seq, idx_t3, energies)

    w_b16 = W_log.astype(jnp.bfloat16)
    b2 = b_log.reshape(1, _V)
    x3b = x3.astype(jnp.bfloat16)
    wclamp = lambda j: (jnp.minimum(j, _NJ - 1), 0)
    bclamp = lambda j: (0, jnp.minimum(j, _NJ - 1))
    logits = pl.pallas_call(
        _proj_body,
        grid=(_NJ + 1,),
        in_specs=[
            whole(),
            whole(),
            pl.BlockSpec((_BV, _H), wclamp),
            pl.BlockSpec((1, _BV), bclamp),
            whole(),
        ],
        out_specs=pl.BlockSpec((_T, _B, _BV), lambda j: (0, 0, j)),
        out_shape=jax.ShapeDtypeStruct((_T, _B, _VO), f32),
        compiler_params=pltpu.CompilerParams(
            dimension_semantics=("parallel",),
            vmem_limit_bytes=60 * 1024 * 1024,
        ),
    )(idx_t, x3b, w_b16, b2, wf)

    return logits, attns, covs, energies
